# Initial kernel scaffold; baseline (speedup 1.0000x reference)
import jax
import jax.numpy as jnp
from jax.experimental import pallas as pl


def kernel(edge_index, edge_type, h_index, r_index, query_table, rel_emb, W, b, W1, b1, W2, b2):
    raise NotImplementedError("write your pallas kernel here")


# speedup vs baseline: 26.0020x; 26.0020x over previous
import functools
import jax
import jax.numpy as jnp
from jax import lax
from jax.experimental import pallas as pl
from jax.experimental.pallas import tpu as pltpu
from jax.experimental.pallas import tpu_sc as plsc

_N = 10000
_D = 128
_E = 160000
_NP = 8
_NTILES = 16
_EPT = _E // _NTILES
_NCH = _EPT // 16
_RPAD = 10240
_DUMMY = 10000
_ZROWS = _RPAD // _NTILES
_OROWS = _N // _NTILES

_mesh = plsc.VectorSubcoreMesh(core_axis_name="c", subcore_axis_name="s")

_GDN = lax.GatherDimensionNumbers(offset_dims=(), collapsed_slice_dims=(0,),
                                  start_index_map=(0,))


def _lane_gather(x, idx):
    return lax.gather(x, idx[:, None], _GDN, slice_sizes=(1,),
                      mode=lax.GatherScatterMode.PROMISE_IN_BOUNDS)


def _any16(x):
    lanes = lax.iota(jnp.int32, 16)
    for sh in (1, 2, 4, 8):
        x = x | _lane_gather(x, (lanes + sh) & 15)
    return x[0]


@functools.partial(
    pl.kernel,
    mesh=_mesh,
    out_type=jax.ShapeDtypeStruct((_NP, _N, _D), jnp.float32),
    scratch_types=[
        pltpu.VMEM((_EPT,), jnp.int32),
        pltpu.VMEM((_EPT,), jnp.int32),
        pltpu.VMEM((_EPT,), jnp.int32),
        pltpu.VMEM((16,), jnp.int32),
        pltpu.VMEM((16, _D), jnp.float32),
        pltpu.VMEM((16,), jnp.int32),
        pltpu.VMEM_SHARED((_RPAD, _D), jnp.float32),
        pltpu.SemaphoreType.DMA,
    ],
)
def _sc_scatter(src_hbm, dst_hbm, typ_hbm, h_hbm, rel_hbm, zero_hbm,
                r_out, src_v, dst_v, typ_v, h_v, stage_v, cnt_v, r_sh, sem):
    c = lax.axis_index("c")
    s = lax.axis_index("s")
    base = s * _EPT
    pltpu.sync_copy(src_hbm.at[pl.ds(base, _EPT)], src_v)
    pltpu.sync_copy(dst_hbm.at[pl.ds(base, _EPT)], dst_v)
    pltpu.sync_copy(typ_hbm.at[pl.ds(base, _EPT)], typ_v)
    pltpu.sync_copy(h_hbm, h_v)
    hv = h_v[...]

    for p_local in range(4):
        p = c * 4 + p_local
        plsc.subcore_barrier()
        pltpu.sync_copy(zero_hbm, r_sh.at[pl.ds(s * _ZROWS, _ZROWS)])
        plsc.subcore_barrier()
        h_splat = _lane_gather(hv, jnp.broadcast_to(p, (16,)))

        def _chunk(i, carry):
            off = pl.multiple_of(i * 16, 16)
            sv = src_v[pl.ds(off, 16)]
            m = sv == h_splat
            hit = _any16(jnp.where(m, 1, 0))

            @pl.when(hit > 0)
            def _():
                dv = dst_v[pl.ds(off, 16)]
                tv = typ_v[pl.ds(off, 16)]
                dpad = jnp.where(m, dv, _DUMMY)
                tpad = jnp.where(m, tv, 0)
                pltpu.async_copy(rel_hbm.at[tpad], stage_v, sem).wait()
                pltpu.sync_copy(stage_v, r_sh.at[dpad], add=True)

            return carry

        lax.fori_loop(0, _NCH, _chunk, 0)
        plsc.subcore_barrier()
        pltpu.sync_copy(r_sh.at[pl.ds(s * 624, 624)],
                        r_out.at[p, pl.ds(s * 624, 624)])

        @pl.when(s == _NTILES - 1)
        def _():
            pltpu.sync_copy(r_sh.at[pl.ds(9984, 16)],
                            r_out.at[p, pl.ds(9984, 16)])


def _tc_body(r1_ref, r2_ref, q_ref, w_ref, b_ref, w1_ref, b1_ref,
             w2_ref, b2_ref, h_ref, out_ref):
    bi = pl.program_id(0)
    blk = pl.program_id(1)
    rows = blk * 1000 + lax.broadcasted_iota(jnp.int32, (1000, 1), 0)

    def score(r_ref, q_row, h_idx):
        q = q_ref[pl.ds(q_row, 1)][0]
        agg = r_ref[0] * q[None, :]
        o = jnp.maximum(jnp.dot(agg, w_ref[...],
                                preferred_element_type=jnp.float32)
                        + b_ref[...], 0.0)
        sel = jnp.where(rows == h_idx, 1.0, 0.0)
        o = o + sel * q[None, :]
        hid = jnp.maximum(jnp.dot(o, w1_ref[...],
                                  preferred_element_type=jnp.float32)
                          + b1_ref[...], 0.0)
        sc = jnp.dot(hid, w2_ref[...],
                     preferred_element_type=jnp.float32) + b2_ref[...]
        return jax.nn.sigmoid(sc)

    s1 = score(r1_ref, 2 * bi, h_ref[2 * bi])
    s2 = score(r2_ref, 2 * bi + 1, h_ref[2 * bi + 1])
    z = s1 * s2
    out_ref[0] = jnp.log((z + 1e-10) / (1.0 - z + 1e-10))


def _tc_mlp(R, q, W, b, W1, b1, W2, b2, h_flat):
    full = lambda shape: pl.BlockSpec(shape, lambda bi, blk: (0,) * len(shape))
    out = pl.pallas_call(
        _tc_body,
        grid=(4, _N // 1000),
        in_specs=[
            pl.BlockSpec((1, 1000, _D), lambda bi, blk: (2 * bi, blk, 0)),
            pl.BlockSpec((1, 1000, _D), lambda bi, blk: (2 * bi + 1, blk, 0)),
            full((_NP, _D)),
            full((_D, _D)),
            full((1, _D)),
            full((_D, _D)),
            full((1, _D)),
            full((_D, 1)),
            full((1, 1)),
            pl.BlockSpec(memory_space=pltpu.SMEM),
        ],
        out_specs=pl.BlockSpec((1, 1000, 1), lambda bi, blk: (bi, blk, 0)),
        out_shape=jax.ShapeDtypeStruct((4, _N, 1), jnp.float32),
    )(R, R, q, W, b, W1, b1, W2, b2, h_flat)
    return out[:, :, 0]


def kernel(edge_index, edge_type, h_index, r_index, query_table, rel_emb,
           W, b, W1, b1, W2, b2):
    src = edge_index[0]
    dst = edge_index[1]
    h_flat = h_index.reshape(_NP).astype(jnp.int32)
    h_pad = jnp.concatenate([h_flat, jnp.zeros((16 - _NP,), jnp.int32)])
    q = query_table[r_index.reshape(_NP)]
    zeros = jnp.zeros((_ZROWS, _D), jnp.float32)
    R = _sc_scatter(src, dst, edge_type.astype(jnp.int32), h_pad, rel_emb,
                    zeros)
    return _tc_mlp(R, q, W, b.reshape(1, _D), W1, b1.reshape(1, _D),
                   W2, b2.reshape(1, 1), h_flat)


# speedup vs baseline: 29.9457x; 1.1517x over previous
import functools
import jax
import jax.numpy as jnp
from jax import lax
from jax.experimental import pallas as pl
from jax.experimental.pallas import tpu as pltpu
from jax.experimental.pallas import tpu_sc as plsc

_N = 10000
_D = 128
_E = 160000
_NP = 8
_NTILES = 16
_EPT = _E // _NTILES
_NCH = _EPT // 16
_RPAD = 10240
_DUMMY = 10000
_ZROWS = _RPAD // _NTILES
_OROWS = _N // _NTILES

_mesh = plsc.VectorSubcoreMesh(core_axis_name="c", subcore_axis_name="s")

_GDN = lax.GatherDimensionNumbers(offset_dims=(), collapsed_slice_dims=(0,),
                                  start_index_map=(0,))


def _lane_gather(x, idx):
    return lax.gather(x, idx[:, None], _GDN, slice_sizes=(1,),
                      mode=lax.GatherScatterMode.PROMISE_IN_BOUNDS)


def _any16(x):
    lanes = lax.iota(jnp.int32, 16)
    for sh in (1, 2, 4, 8):
        x = x | _lane_gather(x, (lanes + sh) & 15)
    return x[0]


_GROUPS = 5
_STEP = 16 * _GROUPS


@functools.partial(
    pl.kernel,
    mesh=_mesh,
    out_type=jax.ShapeDtypeStruct((_NP, _N, _D), jnp.float32),
    scratch_types=[
        pltpu.VMEM((_EPT,), jnp.int32),
        pltpu.VMEM((_EPT,), jnp.int32),
        pltpu.VMEM((_EPT,), jnp.int32),
        pltpu.VMEM((16,), jnp.int32),
        pltpu.VMEM((16, _D), jnp.float32),
        pltpu.VMEM((16,), jnp.int32),
        pltpu.VMEM_SHARED((_RPAD, _D), jnp.float32),
        pltpu.SemaphoreType.DMA,
    ],
)
def _sc_scatter(src_hbm, dst_hbm, typ_hbm, h_hbm, rel_hbm, zero_hbm,
                r_out, src_v, dst_v, typ_v, h_v, stage_v, cnt_v, r_sh, sem):
    c = lax.axis_index("c")
    s = lax.axis_index("s")
    base = s * _EPT
    pltpu.sync_copy(src_hbm.at[pl.ds(base, _EPT)], src_v)
    pltpu.sync_copy(dst_hbm.at[pl.ds(base, _EPT)], dst_v)
    pltpu.sync_copy(typ_hbm.at[pl.ds(base, _EPT)], typ_v)
    pltpu.sync_copy(h_hbm, h_v)
    hv = h_v[...]

    for p_local in range(4):
        p = c * 4 + p_local
        plsc.subcore_barrier()
        pltpu.sync_copy(zero_hbm, r_sh.at[pl.ds(s * _ZROWS, _ZROWS)])
        plsc.subcore_barrier()
        h_splat = _lane_gather(hv, jnp.broadcast_to(p, (16,)))

        def _chunk(i, carry):
            base_off = pl.multiple_of(i * _STEP, 16)
            masks = []
            acc = jnp.zeros((16,), jnp.int32)
            for g in range(_GROUPS):
                sv = src_v[pl.ds(base_off + g * 16, 16)]
                m = sv == h_splat
                masks.append(m)
                acc = acc | jnp.where(m, 1, 0)

            @pl.when(_any16(acc) > 0)
            def _():
                for g in range(_GROUPS):
                    m = masks[g]

                    @pl.when(_any16(jnp.where(m, 1, 0)) > 0)
                    def _(g=g, m=m):
                        off = pl.multiple_of(base_off + g * 16, 16)
                        dv = dst_v[pl.ds(off, 16)]
                        tv = typ_v[pl.ds(off, 16)]
                        dpad = jnp.where(m, dv, _DUMMY)
                        tpad = jnp.where(m, tv, 0)
                        pltpu.async_copy(rel_hbm.at[tpad], stage_v,
                                         sem).wait()
                        pltpu.sync_copy(stage_v, r_sh.at[dpad], add=True)

            return carry

        lax.fori_loop(0, _EPT // _STEP, _chunk, 0)
        plsc.subcore_barrier()
        pltpu.sync_copy(r_sh.at[pl.ds(s * 624, 624)],
                        r_out.at[p, pl.ds(s * 624, 624)])

        @pl.when(s == _NTILES - 1)
        def _():
            pltpu.sync_copy(r_sh.at[pl.ds(9984, 16)],
                            r_out.at[p, pl.ds(9984, 16)])


def _tc_body(r1_ref, r2_ref, q_ref, w_ref, b_ref, w1_ref, b1_ref,
             w2_ref, b2_ref, h_ref, out_ref):
    bi = pl.program_id(0)
    blk = pl.program_id(1)
    rows = blk * 1000 + lax.broadcasted_iota(jnp.int32, (1000, 1), 0)

    def score(r_ref, q_row, h_idx):
        q = q_ref[pl.ds(q_row, 1)][0]
        agg = r_ref[0] * q[None, :]
        o = jnp.maximum(jnp.dot(agg, w_ref[...],
                                preferred_element_type=jnp.float32)
                        + b_ref[...], 0.0)
        sel = jnp.where(rows == h_idx, 1.0, 0.0)
        o = o + sel * q[None, :]
        hid = jnp.maximum(jnp.dot(o, w1_ref[...],
                                  preferred_element_type=jnp.float32)
                          + b1_ref[...], 0.0)
        sc = jnp.dot(hid, w2_ref[...],
                     preferred_element_type=jnp.float32) + b2_ref[...]
        return jax.nn.sigmoid(sc)

    s1 = score(r1_ref, 2 * bi, h_ref[2 * bi])
    s2 = score(r2_ref, 2 * bi + 1, h_ref[2 * bi + 1])
    z = s1 * s2
    out_ref[0] = jnp.log((z + 1e-10) / (1.0 - z + 1e-10))


def _tc_mlp(R, q, W, b, W1, b1, W2, b2, h_flat):
    full = lambda shape: pl.BlockSpec(shape, lambda bi, blk: (0,) * len(shape))
    out = pl.pallas_call(
        _tc_body,
        grid=(4, _N // 1000),
        in_specs=[
            pl.BlockSpec((1, 1000, _D), lambda bi, blk: (2 * bi, blk, 0)),
            pl.BlockSpec((1, 1000, _D), lambda bi, blk: (2 * bi + 1, blk, 0)),
            full((_NP, _D)),
            full((_D, _D)),
            full((1, _D)),
            full((_D, _D)),
            full((1, _D)),
            full((_D, 1)),
            full((1, 1)),
            pl.BlockSpec(memory_space=pltpu.SMEM),
        ],
        out_specs=pl.BlockSpec((1, 1000, 1), lambda bi, blk: (bi, blk, 0)),
        out_shape=jax.ShapeDtypeStruct((4, _N, 1), jnp.float32),
    )(R, R, q, W, b, W1, b1, W2, b2, h_flat)
    return out[:, :, 0]


def kernel(edge_index, edge_type, h_index, r_index, query_table, rel_emb,
           W, b, W1, b1, W2, b2):
    src = edge_index[0]
    dst = edge_index[1]
    h_flat = h_index.reshape(_NP).astype(jnp.int32)
    h_pad = jnp.concatenate([h_flat, jnp.zeros((16 - _NP,), jnp.int32)])
    q = query_table[r_index.reshape(_NP)]
    zeros = jnp.zeros((_ZROWS, _D), jnp.float32)
    R = _sc_scatter(src, dst, edge_type.astype(jnp.int32), h_pad, rel_emb,
                    zeros)
    return _tc_mlp(R, q, W, b.reshape(1, _D), W1, b1.reshape(1, _D),
                   W2, b2.reshape(1, 1), h_flat)


# speedup vs baseline: 30.7943x; 1.0283x over previous
import functools
import jax
import jax.numpy as jnp
from jax import lax
from jax.experimental import pallas as pl
from jax.experimental.pallas import tpu as pltpu
from jax.experimental.pallas import tpu_sc as plsc

_N = 10000
_D = 128
_E = 160000
_NP = 8
_NTILES = 16
_EPT = _E // _NTILES
_NCH = _EPT // 16
_RPAD = 10240
_DUMMY = 10000
_ZROWS = _RPAD // _NTILES
_OROWS = _N // _NTILES

_mesh = plsc.VectorSubcoreMesh(core_axis_name="c", subcore_axis_name="s")

_GDN = lax.GatherDimensionNumbers(offset_dims=(), collapsed_slice_dims=(0,),
                                  start_index_map=(0,))


def _lane_gather(x, idx):
    return lax.gather(x, idx[:, None], _GDN, slice_sizes=(1,),
                      mode=lax.GatherScatterMode.PROMISE_IN_BOUNDS)


def _any16(x):
    lanes = lax.iota(jnp.int32, 16)
    for sh in (1, 2, 4, 8):
        x = x | _lane_gather(x, (lanes + sh) & 15)
    return x[0]


_GROUPS = 25
_STEP = 16 * _GROUPS


@functools.partial(
    pl.kernel,
    mesh=_mesh,
    out_type=jax.ShapeDtypeStruct((_NP, _N, _D), jnp.float32),
    scratch_types=[
        pltpu.VMEM((_EPT,), jnp.int32),
        pltpu.VMEM((_EPT,), jnp.int32),
        pltpu.VMEM((_EPT,), jnp.int32),
        pltpu.VMEM((16,), jnp.int32),
        pltpu.VMEM((16, _D), jnp.float32),
        pltpu.VMEM((16,), jnp.int32),
        pltpu.VMEM_SHARED((_RPAD, _D), jnp.float32),
        pltpu.SemaphoreType.DMA,
    ],
)
def _sc_scatter(src_hbm, dst_hbm, typ_hbm, h_hbm, rel_hbm, zero_hbm,
                r_out, src_v, dst_v, typ_v, h_v, stage_v, cnt_v, r_sh, sem):
    c = lax.axis_index("c")
    s = lax.axis_index("s")
    base = s * _EPT
    pltpu.sync_copy(src_hbm.at[pl.ds(base, _EPT)], src_v)
    pltpu.sync_copy(dst_hbm.at[pl.ds(base, _EPT)], dst_v)
    pltpu.sync_copy(typ_hbm.at[pl.ds(base, _EPT)], typ_v)
    pltpu.sync_copy(h_hbm, h_v)
    hv = h_v[...]

    for p_local in range(4):
        p = c * 4 + p_local
        plsc.subcore_barrier()
        pltpu.sync_copy(zero_hbm, r_sh.at[pl.ds(s * _ZROWS, _ZROWS)])
        plsc.subcore_barrier()
        h_splat = _lane_gather(hv, jnp.broadcast_to(p, (16,)))

        def _chunk(i, carry):
            base_off = pl.multiple_of(i * _STEP, 16)
            masks = []
            acc = jnp.zeros((16,), jnp.int32)
            for g in range(_GROUPS):
                sv = src_v[pl.ds(base_off + g * 16, 16)]
                m = sv == h_splat
                masks.append(m)
                acc = acc | jnp.where(m, 1, 0)

            @pl.when(_any16(acc) > 0)
            def _():
                for g in range(_GROUPS):
                    m = masks[g]

                    @pl.when(_any16(jnp.where(m, 1, 0)) > 0)
                    def _(g=g, m=m):
                        off = pl.multiple_of(base_off + g * 16, 16)
                        dv = dst_v[pl.ds(off, 16)]
                        tv = typ_v[pl.ds(off, 16)]
                        dpad = jnp.where(m, dv, _DUMMY)
                        tpad = jnp.where(m, tv, 0)
                        pltpu.async_copy(rel_hbm.at[tpad], stage_v,
                                         sem).wait()
                        pltpu.sync_copy(stage_v, r_sh.at[dpad], add=True)

            return carry

        lax.fori_loop(0, _EPT // _STEP, _chunk, 0)
        plsc.subcore_barrier()
        pltpu.sync_copy(r_sh.at[pl.ds(s * 624, 624)],
                        r_out.at[p, pl.ds(s * 624, 624)])

        @pl.when(s == _NTILES - 1)
        def _():
            pltpu.sync_copy(r_sh.at[pl.ds(9984, 16)],
                            r_out.at[p, pl.ds(9984, 16)])


def _tc_body(r1_ref, r2_ref, q_ref, w_ref, b_ref, w1_ref, b1_ref,
             w2_ref, b2_ref, h_ref, out_ref):
    bi = pl.program_id(0)
    blk = pl.program_id(1)
    rows = blk * 1000 + lax.broadcasted_iota(jnp.int32, (1000, 1), 0)

    def score(r_ref, q_row, h_idx):
        q = q_ref[pl.ds(q_row, 1)][0]
        agg = r_ref[0] * q[None, :]
        o = jnp.maximum(jnp.dot(agg, w_ref[...],
                                preferred_element_type=jnp.float32)
                        + b_ref[...], 0.0)
        sel = jnp.where(rows == h_idx, 1.0, 0.0)
        o = o + sel * q[None, :]
        hid = jnp.maximum(jnp.dot(o, w1_ref[...],
                                  preferred_element_type=jnp.float32)
                          + b1_ref[...], 0.0)
        sc = jnp.dot(hid, w2_ref[...],
                     preferred_element_type=jnp.float32) + b2_ref[...]
        return jax.nn.sigmoid(sc)

    s1 = score(r1_ref, 2 * bi, h_ref[2 * bi])
    s2 = score(r2_ref, 2 * bi + 1, h_ref[2 * bi + 1])
    z = s1 * s2
    out_ref[0] = jnp.log((z + 1e-10) / (1.0 - z + 1e-10))


def _tc_mlp(R, q, W, b, W1, b1, W2, b2, h_flat):
    full = lambda shape: pl.BlockSpec(shape, lambda bi, blk: (0,) * len(shape))
    out = pl.pallas_call(
        _tc_body,
        grid=(4, _N // 1000),
        in_specs=[
            pl.BlockSpec((1, 1000, _D), lambda bi, blk: (2 * bi, blk, 0)),
            pl.BlockSpec((1, 1000, _D), lambda bi, blk: (2 * bi + 1, blk, 0)),
            full((_NP, _D)),
            full((_D, _D)),
            full((1, _D)),
            full((_D, _D)),
            full((1, _D)),
            full((_D, 1)),
            full((1, 1)),
            pl.BlockSpec(memory_space=pltpu.SMEM),
        ],
        out_specs=pl.BlockSpec((1, 1000, 1), lambda bi, blk: (bi, blk, 0)),
        out_shape=jax.ShapeDtypeStruct((4, _N, 1), jnp.float32),
    )(R, R, q, W, b, W1, b1, W2, b2, h_flat)
    return out[:, :, 0]


def kernel(edge_index, edge_type, h_index, r_index, query_table, rel_emb,
           W, b, W1, b1, W2, b2):
    src = edge_index[0]
    dst = edge_index[1]
    h_flat = h_index.reshape(_NP).astype(jnp.int32)
    h_pad = jnp.concatenate([h_flat, jnp.zeros((16 - _NP,), jnp.int32)])
    q = query_table[r_index.reshape(_NP)]
    zeros = jnp.zeros((_ZROWS, _D), jnp.float32)
    R = _sc_scatter(src, dst, edge_type.astype(jnp.int32), h_pad, rel_emb,
                    zeros)
    return _tc_mlp(R, q, W, b.reshape(1, _D), W1, b1.reshape(1, _D),
                   W2, b2.reshape(1, 1), h_flat)
